# baseline (speedup 1.0000x reference)
import functools

import jax
import jax.numpy as jnp
from jax import lax
from jax.experimental import pallas as pl
from jax.experimental.pallas import tpu as pltpu
from jax.experimental.pallas import tpu_sc as plsc

HID = 64
NC, NS, L = 2, 16, 16
NW = NC * NS

B, F, W, PL = 1024, 26, 4, 10
WALK = W * PL
M = B * F
ROWS_PER_W = M // NW
FC = 13
IDX_C = FC * WALK
NGRP = IDX_C // L
CHUNKS = ROWS_PER_W // FC
NBUF = 2


def _grp_base(g):
  return g * L if g < NGRP else IDX_C - L


def _make_kernel():
  mesh = plsc.VectorSubcoreMesh(core_axis_name="c", subcore_axis_name="s")

  @functools.partial(
      pl.kernel,
      mesh=mesh,
      compiler_params=pltpu.CompilerParams(
          use_tc_tiling_on_sc=False, needs_layout_passes=False),
      out_type=jax.ShapeDtypeStruct((M, HID), jnp.float32),
      scratch_types=[
          [pltpu.VMEM((FC, W, PL), jnp.int32) for _ in range(NBUF)],
          [pltpu.VMEM((IDX_C,), jnp.int32) for _ in range(NBUF)],
          [pltpu.VMEM((IDX_C, HID), jnp.float32) for _ in range(NBUF)],
          [pltpu.VMEM((FC, HID), jnp.float32) for _ in range(NBUF)],
          [pltpu.VMEM(((NGRP + 1) * L,), jnp.int32) for _ in range(3)],
          [pltpu.SemaphoreType.DMA for _ in range(NBUF)],
          [pltpu.SemaphoreType.DMA for _ in range(NBUF)],
      ],
  )
  def body(idx_hbm, table_hbm, out_hbm, slab_bufs, idx_bufs, rows_bufs,
           acc_bufs, coord_tabs, gsems, osems):
    wid = lax.axis_index("s") * NC + lax.axis_index("c")
    row_base = wid * ROWS_PER_W
    ftab, wtab, ptab = coord_tabs

    iota = lax.iota(jnp.int32, L)
    for g in range(NGRP + 1):
      k = iota + _grp_base(g)
      f = (k.astype(jnp.float32) * (1.0 / WALK)).astype(jnp.int32)
      t = k - f * WALK
      w = (t.astype(jnp.float32) * (1.0 / PL)).astype(jnp.int32)
      p = t - w * PL
      ftab[pl.ds(g * L, L)] = f
      wtab[pl.ds(g * L, L)] = w
      ptab[pl.ds(g * L, L)] = p

    def start_gather(ci, bf):
      slab = slab_bufs[bf]
      idx1 = idx_bufs[bf]
      pltpu.sync_copy(idx_hbm.at[pl.ds(row_base + ci * FC, FC)], slab)
      for g in range(NGRP + 1):
        base = _grp_base(g)
        f = ftab[pl.ds(g * L, L)]
        w = wtab[pl.ds(g * L, L)]
        p = ptab[pl.ds(g * L, L)]
        idx1[pl.ds(base, L)] = plsc.load_gather(slab, [f, w, p])
      pltpu.async_copy(table_hbm.at[idx1], rows_bufs[bf], gsems[bf])

    def wait_gather(bf):
      pltpu.make_async_copy(
          table_hbm.at[idx_bufs[bf]], rows_bufs[bf], gsems[bf]).wait()

    def out_slice(ci):
      return out_hbm.at[pl.ds(row_base + ci * FC, FC), :]

    start_gather(0, 0)

    def outer(ci2, _):
      base_ci = ci2 * NBUF
      for bf in range(NBUF):
        ci = base_ci + bf
        nbf = (bf + 1) % NBUF

        @pl.when(ci + 1 < CHUNKS)
        def _():
          start_gather(ci + 1, nbf)

        wait_gather(bf)
        rows_v = rows_bufs[bf]
        acc_v = acc_bufs[bf]

        @pl.when(ci2 > 0)
        def _():
          pltpu.make_async_copy(acc_v, out_slice(ci), osems[bf]).wait()

        for r in range(FC):
          def red_body(jo, carry):
            a0, a1, a2, a3 = carry
            for ji in range(4):
              rr = r * WALK + jo * 4 + ji
              a0 = a0 + rows_v[rr, pl.ds(0, L)]
              a1 = a1 + rows_v[rr, pl.ds(L, L)]
              a2 = a2 + rows_v[rr, pl.ds(2 * L, L)]
              a3 = a3 + rows_v[rr, pl.ds(3 * L, L)]
            return (a0, a1, a2, a3)

          z = jnp.zeros((L,), jnp.float32)
          a0, a1, a2, a3 = lax.fori_loop(0, WALK // 4, red_body,
                                         (z, z, z, z))
          acc_v[r, pl.ds(0, L)] = a0
          acc_v[r, pl.ds(L, L)] = a1
          acc_v[r, pl.ds(2 * L, L)] = a2
          acc_v[r, pl.ds(3 * L, L)] = a3
        pltpu.async_copy(acc_v, out_slice(ci), osems[bf])
      return 0

    lax.fori_loop(0, CHUNKS // NBUF, outer, 0)
    for bf in range(NBUF):
      pltpu.make_async_copy(
          acc_bufs[bf], out_slice(CHUNKS - NBUF + bf), osems[bf]).wait()

  return body


_sc_kernel = _make_kernel()


def kernel(walk_paths, node_embeddings, linear_w):
  del linear_w
  out = _sc_kernel(walk_paths.reshape(M, W, PL), node_embeddings)
  return out.reshape(B, F, HID)


# speedup vs baseline: 1.0013x; 1.0013x over previous
import functools

import jax
import jax.numpy as jnp
from jax import lax
from jax.experimental import pallas as pl
from jax.experimental.pallas import tpu as pltpu
from jax.experimental.pallas import tpu_sc as plsc

HID = 64
NC, NS, L = 2, 16, 16
NW = NC * NS

B, F, W, PL = 1024, 26, 4, 10
WALK = W * PL
M = B * F
ROWS_PER_W = M // NW
FC = 13
IDX_C = FC * WALK
NGRP = IDX_C // L
CHUNKS = ROWS_PER_W // FC
NBUF = 2


def _grp_base(g):
  return g * L if g < NGRP else IDX_C - L


def _make_kernel():
  mesh = plsc.VectorSubcoreMesh(core_axis_name="c", subcore_axis_name="s")

  @functools.partial(
      pl.kernel,
      mesh=mesh,
      compiler_params=pltpu.CompilerParams(
          use_tc_tiling_on_sc=False, needs_layout_passes=False),
      out_type=jax.ShapeDtypeStruct((M, HID), jnp.float32),
      scratch_types=[
          [pltpu.VMEM((FC, W, PL), jnp.int32) for _ in range(NBUF)],
          [pltpu.VMEM((IDX_C,), jnp.int32) for _ in range(NBUF)],
          [pltpu.VMEM((IDX_C, HID), jnp.float32) for _ in range(NBUF)],
          [pltpu.VMEM((FC, HID), jnp.float32) for _ in range(NBUF)],
          [pltpu.VMEM(((NGRP + 1) * L,), jnp.int32) for _ in range(3)],
          [pltpu.SemaphoreType.DMA for _ in range(NBUF)],
          [pltpu.SemaphoreType.DMA for _ in range(NBUF)],
      ],
  )
  def body(idx_hbm, table_hbm, out_hbm, slab_bufs, idx_bufs, rows_bufs,
           acc_bufs, coord_tabs, gsems, osems):
    wid = lax.axis_index("s") * NC + lax.axis_index("c")
    b_base = wid * (B // NW)
    ftab, wtab, ptab = coord_tabs

    iota = lax.iota(jnp.int32, L)
    for g in range(NGRP + 1):
      k = iota + _grp_base(g)
      f = (k.astype(jnp.float32) * (1.0 / WALK)).astype(jnp.int32)
      t = k - f * WALK
      w = (t.astype(jnp.float32) * (1.0 / PL)).astype(jnp.int32)
      p = t - w * PL
      ftab[pl.ds(g * L, L)] = f
      wtab[pl.ds(g * L, L)] = w
      ptab[pl.ds(g * L, L)] = p

    def start_gather(b, half, bf):
      slab = slab_bufs[bf]
      idx1 = idx_bufs[bf]
      pltpu.sync_copy(idx_hbm.at[b, pl.ds(half * FC, FC)], slab)
      for g in range(NGRP + 1):
        base = _grp_base(g)
        f = ftab[pl.ds(g * L, L)]
        w = wtab[pl.ds(g * L, L)]
        p = ptab[pl.ds(g * L, L)]
        idx1[pl.ds(base, L)] = plsc.load_gather(slab, [f, w, p])
      pltpu.async_copy(table_hbm.at[idx1], rows_bufs[bf], gsems[bf])

    def wait_gather(bf):
      pltpu.make_async_copy(
          table_hbm.at[idx_bufs[bf]], rows_bufs[bf], gsems[bf]).wait()

    def out_slice(b, half):
      return out_hbm.at[pl.ds(b * F + half * FC, FC), :]

    start_gather(b_base, 0, 0)

    def outer(ib, _):
      b = b_base + ib
      for half in range(2):
        bf = half
        nbf = (half + 1) % NBUF

        if half == 0:
          start_gather(b, 1, nbf)
        else:
          @pl.when(ib + 1 < B // NW)
          def _():
            start_gather(b + 1, 0, nbf)

        wait_gather(bf)
        rows_v = rows_bufs[bf]
        acc_v = acc_bufs[bf]

        @pl.when(ib > 0)
        def _():
          pltpu.make_async_copy(acc_v, out_slice(b, half), osems[bf]).wait()

        for r in range(FC):
          def red_body(jo, carry):
            a0, a1, a2, a3 = carry
            for ji in range(4):
              rr = r * WALK + jo * 4 + ji
              a0 = a0 + rows_v[rr, pl.ds(0, L)]
              a1 = a1 + rows_v[rr, pl.ds(L, L)]
              a2 = a2 + rows_v[rr, pl.ds(2 * L, L)]
              a3 = a3 + rows_v[rr, pl.ds(3 * L, L)]
            return (a0, a1, a2, a3)

          z = jnp.zeros((L,), jnp.float32)
          a0, a1, a2, a3 = lax.fori_loop(0, WALK // 4, red_body,
                                         (z, z, z, z))
          acc_v[r, pl.ds(0, L)] = a0
          acc_v[r, pl.ds(L, L)] = a1
          acc_v[r, pl.ds(2 * L, L)] = a2
          acc_v[r, pl.ds(3 * L, L)] = a3
        pltpu.async_copy(acc_v, out_slice(b, half), osems[bf])
      return 0

    lax.fori_loop(0, B // NW, outer, 0)
    b_last = b_base + B // NW - 1
    for bf in range(NBUF):
      pltpu.make_async_copy(
          acc_bufs[bf], out_slice(b_last, bf), osems[bf]).wait()

  return body


_sc_kernel = _make_kernel()


def kernel(walk_paths, node_embeddings, linear_w):
  del linear_w
  out = _sc_kernel(walk_paths, node_embeddings)
  return out.reshape(B, F, HID)


# speedup vs baseline: 1.0470x; 1.0456x over previous
import functools

import jax
import jax.numpy as jnp
from jax import lax
from jax.experimental import pallas as pl
from jax.experimental.pallas import tpu as pltpu
from jax.experimental.pallas import tpu_sc as plsc

HID = 64
NC, NS, L = 2, 16, 16
NW = NC * NS

B, F, W, PL = 1024, 26, 4, 10
WALK = W * PL
M = B * F
ROWS_PER_W = M // NW
FC = 13
IDX_C = FC * WALK
NGRP = IDX_C // L
CHUNKS = ROWS_PER_W // FC
NBUF = 2


def _grp_base(g):
  return g * L if g < NGRP else IDX_C - L


def _make_kernel():
  mesh = plsc.VectorSubcoreMesh(core_axis_name="c", subcore_axis_name="s")

  @functools.partial(
      pl.kernel,
      mesh=mesh,
      compiler_params=pltpu.CompilerParams(
          use_tc_tiling_on_sc=False, needs_layout_passes=False),
      out_type=jax.ShapeDtypeStruct((M, HID), jnp.float32),
      scratch_types=[
          [pltpu.VMEM((FC, WALK), jnp.int32) for _ in range(NBUF)],
          [pltpu.VMEM((IDX_C,), jnp.int32) for _ in range(NBUF)],
          [pltpu.VMEM((IDX_C, HID), jnp.float32) for _ in range(NBUF)],
          [pltpu.VMEM((FC, HID), jnp.float32) for _ in range(NBUF)],
          [pltpu.VMEM(((NGRP + 1) * L,), jnp.int32) for _ in range(3)],
          [pltpu.SemaphoreType.DMA for _ in range(NBUF)],
          [pltpu.SemaphoreType.DMA for _ in range(NBUF)],
      ],
  )
  def body(idx_hbm, table_hbm, out_hbm, slab_bufs, idx_bufs, rows_bufs,
           acc_bufs, coord_tabs, gsems, osems):
    wid = lax.axis_index("s") * NC + lax.axis_index("c")
    b_base = wid * (B // NW)
    ftab, wtab, ptab = coord_tabs

    iota = lax.iota(jnp.int32, L)
    for g in range(NGRP + 1):
      k = iota + _grp_base(g)
      f = (k.astype(jnp.float32) * (1.0 / WALK)).astype(jnp.int32)
      t = k - f * WALK
      ftab[pl.ds(g * L, L)] = f
      wtab[pl.ds(g * L, L)] = t

    def start_gather(b, half, bf):
      slab = slab_bufs[bf]
      idx1 = idx_bufs[bf]
      pltpu.sync_copy(
          idx_hbm.at[pl.ds(b * F + half * FC, FC)], slab)
      for g in range(NGRP + 1):
        base = _grp_base(g)
        f = ftab[pl.ds(g * L, L)]
        t = wtab[pl.ds(g * L, L)]
        idx1[pl.ds(base, L)] = plsc.load_gather(slab, [f, t])
      pltpu.async_copy(table_hbm.at[idx1], rows_bufs[bf], gsems[bf])

    def wait_gather(bf):
      pltpu.make_async_copy(
          table_hbm.at[idx_bufs[bf]], rows_bufs[bf], gsems[bf]).wait()

    def out_slice(b, half):
      return out_hbm.at[pl.ds(b * F + half * FC, FC), :]

    start_gather(b_base, 0, 0)

    def outer(ib, _):
      b = b_base + ib
      for half in range(2):
        bf = half
        nbf = (half + 1) % NBUF

        if half == 0:
          start_gather(b, 1, nbf)
        else:
          @pl.when(ib + 1 < B // NW)
          def _():
            start_gather(b + 1, 0, nbf)

        wait_gather(bf)
        rows_v = rows_bufs[bf]
        acc_v = acc_bufs[bf]

        @pl.when(ib > 0)
        def _():
          pltpu.make_async_copy(acc_v, out_slice(b, half), osems[bf]).wait()

        for r in range(FC):
          def red_body(jo, carry):
            a0, a1, a2, a3 = carry
            for ji in range(4):
              rr = r * WALK + jo * 4 + ji
              a0 = a0 + rows_v[rr, pl.ds(0, L)]
              a1 = a1 + rows_v[rr, pl.ds(L, L)]
              a2 = a2 + rows_v[rr, pl.ds(2 * L, L)]
              a3 = a3 + rows_v[rr, pl.ds(3 * L, L)]
            return (a0, a1, a2, a3)

          z = jnp.zeros((L,), jnp.float32)
          a0, a1, a2, a3 = lax.fori_loop(0, WALK // 4, red_body,
                                         (z, z, z, z))
          acc_v[r, pl.ds(0, L)] = a0
          acc_v[r, pl.ds(L, L)] = a1
          acc_v[r, pl.ds(2 * L, L)] = a2
          acc_v[r, pl.ds(3 * L, L)] = a3
        pltpu.async_copy(acc_v, out_slice(b, half), osems[bf])
      return 0

    lax.fori_loop(0, B // NW, outer, 0)
    b_last = b_base + B // NW - 1
    for bf in range(NBUF):
      pltpu.make_async_copy(
          acc_bufs[bf], out_slice(b_last, bf), osems[bf]).wait()

  return body


_sc_kernel = _make_kernel()


def kernel(walk_paths, node_embeddings, linear_w):
  del linear_w
  out = _sc_kernel(walk_paths.reshape(M, WALK), node_embeddings)
  return out.reshape(B, F, HID)


# speedup vs baseline: 1.0544x; 1.0071x over previous
import functools

import jax
import jax.numpy as jnp
from jax import lax
from jax.experimental import pallas as pl
from jax.experimental.pallas import tpu as pltpu
from jax.experimental.pallas import tpu_sc as plsc

HID = 64
NC, NS, L = 2, 16, 16
NW = NC * NS

B, F, W, PL = 1024, 26, 4, 10
WALK = W * PL
M = B * F
ROWS_PER_W = M // NW

RK = 104
K1_CHUNKS = ROWS_PER_W // RK
NGRP1 = RK * WALK // L

C = 13
IDX_C = C * WALK
CHUNKS = ROWS_PER_W // C
NBUF = 2

_mesh = plsc.VectorSubcoreMesh(core_axis_name="c", subcore_axis_name="s")


def _make_flatten():
  @functools.partial(
      pl.kernel,
      mesh=_mesh,
      compiler_params=pltpu.CompilerParams(
          use_tc_tiling_on_sc=True, needs_layout_passes=False),
      out_type=jax.ShapeDtypeStruct((M * WALK,), jnp.int32),
      scratch_types=[
          pltpu.VMEM((RK, W, PL), jnp.int32),
          pltpu.VMEM((RK * WALK,), jnp.int32),
          [pltpu.VMEM((NGRP1 * L,), jnp.int32) for _ in range(3)],
      ],
  )
  def body(idx_hbm, out_hbm, slab, flat, coord_tabs):
    wid = lax.axis_index("s") * NC + lax.axis_index("c")
    row_base = wid * ROWS_PER_W
    ftab, wtab, ptab = coord_tabs

    iota = lax.iota(jnp.int32, L)
    for g in range(NGRP1):
      k = iota + g * L
      f = (k.astype(jnp.float32) * (1.0 / WALK)).astype(jnp.int32)
      t = k - f * WALK
      w = (t.astype(jnp.float32) * (1.0 / PL)).astype(jnp.int32)
      p = t - w * PL
      ftab[pl.ds(g * L, L)] = f
      wtab[pl.ds(g * L, L)] = w
      ptab[pl.ds(g * L, L)] = p

    def chunk(ci, _):
      r0 = row_base + ci * RK
      pltpu.sync_copy(idx_hbm.at[pl.ds(r0, RK)], slab)
      for g in range(NGRP1):
        f = ftab[pl.ds(g * L, L)]
        w = wtab[pl.ds(g * L, L)]
        p = ptab[pl.ds(g * L, L)]
        flat[pl.ds(g * L, L)] = plsc.load_gather(slab, [f, w, p])
      pltpu.sync_copy(flat, out_hbm.at[pl.ds(r0 * WALK, RK * WALK)])
      return 0

    lax.fori_loop(0, K1_CHUNKS, chunk, 0)

  return body


def _make_gather_reduce():
  @functools.partial(
      pl.kernel,
      mesh=_mesh,
      compiler_params=pltpu.CompilerParams(
          use_tc_tiling_on_sc=False, needs_layout_passes=False),
      out_type=jax.ShapeDtypeStruct((M, HID), jnp.float32),
      scratch_types=[
          [pltpu.VMEM((IDX_C,), jnp.int32) for _ in range(NBUF)],
          [pltpu.VMEM((IDX_C, HID), jnp.float32) for _ in range(NBUF)],
          [pltpu.VMEM((C, HID), jnp.float32) for _ in range(NBUF)],
          [pltpu.SemaphoreType.DMA for _ in range(NBUF)],
          [pltpu.SemaphoreType.DMA for _ in range(NBUF)],
      ],
  )
  def body(idx_hbm, table_hbm, out_hbm, idx_bufs, rows_bufs, acc_bufs,
           gsems, osems):
    wid = lax.axis_index("s") * NC + lax.axis_index("c")
    row_base = wid * ROWS_PER_W

    def start_gather(ci, bf):
      pltpu.sync_copy(
          idx_hbm.at[pl.ds((row_base + ci * C) * WALK, IDX_C)],
          idx_bufs[bf])
      pltpu.async_copy(table_hbm.at[idx_bufs[bf]], rows_bufs[bf], gsems[bf])

    def wait_gather(bf):
      pltpu.make_async_copy(
          table_hbm.at[idx_bufs[bf]], rows_bufs[bf], gsems[bf]).wait()

    def out_slice(ci):
      return out_hbm.at[pl.ds(row_base + ci * C, C), :]

    start_gather(0, 0)

    def outer(ci2, _):
      base_ci = ci2 * NBUF
      for bf in range(NBUF):
        ci = base_ci + bf
        nbf = (bf + 1) % NBUF

        @pl.when(ci + 1 < CHUNKS)
        def _():
          start_gather(ci + 1, nbf)

        wait_gather(bf)
        rows_v = rows_bufs[bf]
        acc_v = acc_bufs[bf]

        @pl.when(ci2 > 0)
        def _():
          pltpu.make_async_copy(acc_v, out_slice(ci), osems[bf]).wait()

        for r in range(C):
          def red_body(jo, carry):
            a0, a1, a2, a3 = carry
            for ji in range(4):
              rr = r * WALK + jo * 4 + ji
              a0 = a0 + rows_v[rr, pl.ds(0, L)]
              a1 = a1 + rows_v[rr, pl.ds(L, L)]
              a2 = a2 + rows_v[rr, pl.ds(2 * L, L)]
              a3 = a3 + rows_v[rr, pl.ds(3 * L, L)]
            return (a0, a1, a2, a3)

          z = jnp.zeros((L,), jnp.float32)
          a0, a1, a2, a3 = lax.fori_loop(0, WALK // 4, red_body,
                                         (z, z, z, z))
          acc_v[r, pl.ds(0, L)] = a0
          acc_v[r, pl.ds(L, L)] = a1
          acc_v[r, pl.ds(2 * L, L)] = a2
          acc_v[r, pl.ds(3 * L, L)] = a3
        pltpu.async_copy(acc_v, out_slice(ci), osems[bf])
      return 0

    lax.fori_loop(0, CHUNKS // NBUF, outer, 0)
    for bf in range(NBUF):
      pltpu.make_async_copy(
          acc_bufs[bf], out_slice(CHUNKS - NBUF + bf), osems[bf]).wait()

  return body


_flatten = _make_flatten()
_gather_reduce = _make_gather_reduce()


def kernel(walk_paths, node_embeddings, linear_w):
  del linear_w
  flat_idx = _flatten(walk_paths.reshape(M, W, PL))
  out = _gather_reduce(flat_idx, node_embeddings)
  return out.reshape(B, F, HID)


# speedup vs baseline: 1.0550x; 1.0005x over previous
import functools

import jax
import jax.numpy as jnp
from jax import lax
from jax.experimental import pallas as pl
from jax.experimental.pallas import tpu as pltpu
from jax.experimental.pallas import tpu_sc as plsc

HID = 64
NC, NS, L = 2, 16, 16
NW = NC * NS

B, F, W, PL = 1024, 26, 4, 10
WALK = W * PL
M = B * F
ROWS_PER_W = M // NW

NB1 = 4
ROWS_B = F * WALK
K1_CHUNKS = B // NW // NB1
NGRP1 = NB1 * ROWS_B // L

C = 13
IDX_C = C * WALK
CHUNKS = ROWS_PER_W // C
NBUF = 2

_mesh = plsc.VectorSubcoreMesh(core_axis_name="c", subcore_axis_name="s")


def _make_flatten():
  @functools.partial(
      pl.kernel,
      mesh=_mesh,
      compiler_params=pltpu.CompilerParams(
          use_tc_tiling_on_sc=True, needs_layout_passes=False),
      out_type=jax.ShapeDtypeStruct((M * WALK,), jnp.int32),
      scratch_types=[
          pltpu.VMEM((NB1, F, W, PL), jnp.int32),
          pltpu.VMEM((NB1 * ROWS_B,), jnp.int32),
          [pltpu.VMEM((NGRP1 * L,), jnp.int32) for _ in range(4)],
      ],
  )
  def body(idx_hbm, out_hbm, slab, flat, coord_tabs):
    wid = lax.axis_index("s") * NC + lax.axis_index("c")
    b_base = wid * (B // NW)
    btab, ftab, wtab, ptab = coord_tabs

    iota = lax.iota(jnp.int32, L)
    for g in range(NGRP1):
      bloc = (g * L) // ROWS_B
      kk = iota + (g * L - bloc * ROWS_B)
      f = (kk.astype(jnp.float32) * (1.0 / WALK)).astype(jnp.int32)
      t = kk - f * WALK
      w = (t.astype(jnp.float32) * (1.0 / PL)).astype(jnp.int32)
      p = t - w * PL
      btab[pl.ds(g * L, L)] = iota * 0 + bloc
      ftab[pl.ds(g * L, L)] = f
      wtab[pl.ds(g * L, L)] = w
      ptab[pl.ds(g * L, L)] = p

    def chunk(ci, _):
      b0 = b_base + ci * NB1
      pltpu.sync_copy(idx_hbm.at[pl.ds(b0, NB1)], slab)
      for g in range(NGRP1):
        bv = btab[pl.ds(g * L, L)]
        f = ftab[pl.ds(g * L, L)]
        w = wtab[pl.ds(g * L, L)]
        p = ptab[pl.ds(g * L, L)]
        flat[pl.ds(g * L, L)] = plsc.load_gather(slab, [bv, f, w, p])
      pltpu.sync_copy(flat, out_hbm.at[pl.ds(b0 * ROWS_B, NB1 * ROWS_B)])
      return 0

    lax.fori_loop(0, K1_CHUNKS, chunk, 0)

  return body


def _make_gather_reduce():
  @functools.partial(
      pl.kernel,
      mesh=_mesh,
      compiler_params=pltpu.CompilerParams(
          use_tc_tiling_on_sc=False, needs_layout_passes=False),
      out_type=jax.ShapeDtypeStruct((M, HID), jnp.float32),
      scratch_types=[
          [pltpu.VMEM((IDX_C,), jnp.int32) for _ in range(NBUF)],
          [pltpu.VMEM((IDX_C, HID), jnp.float32) for _ in range(NBUF)],
          [pltpu.VMEM((C, HID), jnp.float32) for _ in range(NBUF)],
          [pltpu.SemaphoreType.DMA for _ in range(NBUF)],
          [pltpu.SemaphoreType.DMA for _ in range(NBUF)],
      ],
  )
  def body(idx_hbm, table_hbm, out_hbm, idx_bufs, rows_bufs, acc_bufs,
           gsems, osems):
    wid = lax.axis_index("s") * NC + lax.axis_index("c")
    row_base = wid * ROWS_PER_W

    def start_gather(ci, bf):
      pltpu.sync_copy(
          idx_hbm.at[pl.ds((row_base + ci * C) * WALK, IDX_C)],
          idx_bufs[bf])
      pltpu.async_copy(table_hbm.at[idx_bufs[bf]], rows_bufs[bf], gsems[bf])

    def wait_gather(bf):
      pltpu.make_async_copy(
          table_hbm.at[idx_bufs[bf]], rows_bufs[bf], gsems[bf]).wait()

    def out_slice(ci):
      return out_hbm.at[pl.ds(row_base + ci * C, C), :]

    start_gather(0, 0)

    def outer(ci2, _):
      base_ci = ci2 * NBUF
      for bf in range(NBUF):
        ci = base_ci + bf
        nbf = (bf + 1) % NBUF

        @pl.when(ci + 1 < CHUNKS)
        def _():
          start_gather(ci + 1, nbf)

        wait_gather(bf)
        rows_v = rows_bufs[bf]
        acc_v = acc_bufs[bf]

        @pl.when(ci2 > 0)
        def _():
          pltpu.make_async_copy(acc_v, out_slice(ci), osems[bf]).wait()

        for r in range(C):
          def red_body(jo, carry):
            a0, a1, a2, a3 = carry
            for ji in range(4):
              rr = r * WALK + jo * 4 + ji
              a0 = a0 + rows_v[rr, pl.ds(0, L)]
              a1 = a1 + rows_v[rr, pl.ds(L, L)]
              a2 = a2 + rows_v[rr, pl.ds(2 * L, L)]
              a3 = a3 + rows_v[rr, pl.ds(3 * L, L)]
            return (a0, a1, a2, a3)

          z = jnp.zeros((L,), jnp.float32)
          a0, a1, a2, a3 = lax.fori_loop(0, WALK // 4, red_body,
                                         (z, z, z, z))
          acc_v[r, pl.ds(0, L)] = a0
          acc_v[r, pl.ds(L, L)] = a1
          acc_v[r, pl.ds(2 * L, L)] = a2
          acc_v[r, pl.ds(3 * L, L)] = a3
        pltpu.async_copy(acc_v, out_slice(ci), osems[bf])
      return 0

    lax.fori_loop(0, CHUNKS // NBUF, outer, 0)
    for bf in range(NBUF):
      pltpu.make_async_copy(
          acc_bufs[bf], out_slice(CHUNKS - NBUF + bf), osems[bf]).wait()

  return body


_flatten = _make_flatten()
_gather_reduce = _make_gather_reduce()


def kernel(walk_paths, node_embeddings, linear_w):
  del linear_w
  flat_idx = _flatten(walk_paths)
  out = _gather_reduce(flat_idx, node_embeddings)
  return out.reshape(B, F, HID)
